# baseline (speedup 1.0000x reference)
import functools

import jax
import jax.numpy as jnp
from jax import lax
from jax.experimental import pallas as pl
from jax.experimental.pallas import tpu as pltpu
from jax.experimental.pallas import tpu_sc as plsc

NC = 2
NS = 16
LANES = 16
NW = NC * NS
BATCH = 16384
BPW = BATCH // NW
DIM = 32
HALF = DIM // 2
CHUNK = 128
NCHUNK = BPW // CHUNK


def _sc_body(head_h, rel_h, tail_h, ent_h, remb_h, out_h,
             hidx, ridx, tidx, hrows, rrows, trows, scores, sem):
    wid = lax.axis_index("s") * NC + lax.axis_index("c")

    pltpu.sync_copy(head_h.at[wid], hidx)
    pltpu.sync_copy(rel_h.at[wid], ridx)
    pltpu.sync_copy(tail_h.at[wid], tidx)

    copies = []
    for j in range(NCHUNK):
        dst = pl.ds(j * CHUNK, CHUNK)
        copies.append(pltpu.async_copy(ent_h.at[hidx.at[j]], hrows.at[dst], sem))
        copies.append(pltpu.async_copy(ent_h.at[tidx.at[j]], trows.at[dst], sem))
        copies.append(pltpu.async_copy(remb_h.at[ridx.at[j]], rrows.at[dst], sem))
    for c in copies:
        c.wait()

    lane_iota = lax.iota(jnp.int32, LANES)
    zero16 = jnp.zeros((LANES,), jnp.float32)
    cols = [jnp.full((LANES,), d, jnp.int32) for d in range(DIM)]

    def block(b, carry):
        rows = b * LANES + lane_iota

        def dim_step(acc, d):
            hre = plsc.load_gather(hrows, [rows, cols[d]])
            him = plsc.load_gather(hrows, [rows, cols[d + HALF]])
            rre = plsc.load_gather(rrows, [rows, cols[d]])
            rim = plsc.load_gather(rrows, [rows, cols[d + HALF]])
            tre = plsc.load_gather(trows, [rows, cols[d]])
            tim = plsc.load_gather(trows, [rows, cols[d + HALF]])
            real = hre * rre - him * rim
            imag = hre * rim + him * rre
            return acc + (tre * real - tim * imag)

        acc = zero16
        for d in range(HALF):
            acc = dim_step(acc, d)
        scores[pl.ds(b * LANES, LANES)] = 1.0 / (1.0 + jnp.exp(-acc))
        return carry

    lax.fori_loop(0, BPW // LANES, block, 0)

    pltpu.sync_copy(scores, out_h.at[wid])


@functools.cache
def _build_sc_kernel():
    return pl.kernel(
        _sc_body,
        out_type=jax.ShapeDtypeStruct((NW, BPW), jnp.float32),
        mesh=plsc.VectorSubcoreMesh(
            core_axis_name="c", subcore_axis_name="s",
            num_cores=NC, num_subcores=NS),
        compiler_params=pltpu.CompilerParams(
            needs_layout_passes=False, use_tc_tiling_on_sc=False),
        scratch_types=[
            pltpu.VMEM((NCHUNK, CHUNK), jnp.int32),
            pltpu.VMEM((NCHUNK, CHUNK), jnp.int32),
            pltpu.VMEM((NCHUNK, CHUNK), jnp.int32),
            pltpu.VMEM((BPW, DIM), jnp.float32),
            pltpu.VMEM((BPW, DIM), jnp.float32),
            pltpu.VMEM((BPW, DIM), jnp.float32),
            pltpu.VMEM((BPW,), jnp.float32),
            pltpu.SemaphoreType.DMA,
        ],
    )


def kernel(head, relation, tail, entity_embed, relation_embed):
    h3 = head.reshape(NW, NCHUNK, CHUNK)
    r3 = relation.reshape(NW, NCHUNK, CHUNK)
    t3 = tail.reshape(NW, NCHUNK, CHUNK)
    out = _build_sc_kernel()(h3, r3, t3, entity_embed, relation_embed)
    return out.reshape(BATCH)


# speedup vs baseline: 1.0019x; 1.0019x over previous
import functools

import jax
import jax.numpy as jnp
from jax import lax
from jax.experimental import pallas as pl
from jax.experimental.pallas import tpu as pltpu
from jax.experimental.pallas import tpu_sc as plsc

NC = 2
NS = 16
LANES = 16
NW = NC * NS
BATCH = 16384
BPW = BATCH // NW
DIM = 32
HALF = DIM // 2
CHUNK = 128
NCHUNK = BPW // CHUNK


def _sc_body(head_h, rel_h, tail_h, ent_h, remb_h, out_h,
             hidx, ridx, tidx, hrows, rrows, trows, scores, sem):
    wid = lax.axis_index("s") * NC + lax.axis_index("c")

    pltpu.sync_copy(head_h.at[wid], hidx)
    pltpu.sync_copy(rel_h.at[wid], ridx)
    pltpu.sync_copy(tail_h.at[wid], tidx)

    copies = []
    for j in range(NCHUNK):
        dst = pl.ds(j * CHUNK, CHUNK)
        copies.append(
            pltpu.async_copy(ent_h.at[hidx.at[j]], hrows.at[dst], sem.at[3 * j]))
        copies.append(
            pltpu.async_copy(ent_h.at[tidx.at[j]], trows.at[dst], sem.at[3 * j + 1]))
        copies.append(
            pltpu.async_copy(remb_h.at[ridx.at[j]], rrows.at[dst], sem.at[3 * j + 2]))
    for c in copies:
        c.wait()

    lane_iota = lax.iota(jnp.int32, LANES)
    zero16 = jnp.zeros((LANES,), jnp.float32)
    cols = [jnp.full((LANES,), d, jnp.int32) for d in range(DIM)]

    def block(b, carry):
        rows = b * LANES + lane_iota

        def dim_step(acc, d):
            hre = plsc.load_gather(hrows, [rows, cols[d]])
            him = plsc.load_gather(hrows, [rows, cols[d + HALF]])
            rre = plsc.load_gather(rrows, [rows, cols[d]])
            rim = plsc.load_gather(rrows, [rows, cols[d + HALF]])
            tre = plsc.load_gather(trows, [rows, cols[d]])
            tim = plsc.load_gather(trows, [rows, cols[d + HALF]])
            real = hre * rre - him * rim
            imag = hre * rim + him * rre
            return acc + (tre * real - tim * imag)

        acc = zero16
        for d in range(HALF):
            acc = dim_step(acc, d)
        scores[pl.ds(b * LANES, LANES)] = 1.0 / (1.0 + jnp.exp(-acc))
        return carry

    lax.fori_loop(0, BPW // LANES, block, 0)

    pltpu.sync_copy(scores, out_h.at[wid])


@functools.cache
def _build_sc_kernel():
    return pl.kernel(
        _sc_body,
        out_type=jax.ShapeDtypeStruct((NW, BPW), jnp.float32),
        mesh=plsc.VectorSubcoreMesh(
            core_axis_name="c", subcore_axis_name="s",
            num_cores=NC, num_subcores=NS),
        compiler_params=pltpu.CompilerParams(
            needs_layout_passes=False, use_tc_tiling_on_sc=False),
        scratch_types=[
            pltpu.VMEM((NCHUNK, CHUNK), jnp.int32),
            pltpu.VMEM((NCHUNK, CHUNK), jnp.int32),
            pltpu.VMEM((NCHUNK, CHUNK), jnp.int32),
            pltpu.VMEM((BPW, DIM), jnp.float32),
            pltpu.VMEM((BPW, DIM), jnp.float32),
            pltpu.VMEM((BPW, DIM), jnp.float32),
            pltpu.VMEM((BPW,), jnp.float32),
            pltpu.SemaphoreType.DMA((3 * NCHUNK,)),
        ],
    )


def kernel(head, relation, tail, entity_embed, relation_embed):
    h3 = head.reshape(NW, NCHUNK, CHUNK)
    r3 = relation.reshape(NW, NCHUNK, CHUNK)
    t3 = tail.reshape(NW, NCHUNK, CHUNK)
    out = _build_sc_kernel()(h3, r3, t3, entity_embed, relation_embed)
    return out.reshape(BATCH)
